# Initial kernel scaffold; baseline (speedup 1.0000x reference)
import jax
import jax.numpy as jnp
from jax.experimental import pallas as pl


def kernel(points, features, params):
    raise NotImplementedError("write your pallas kernel here")


# speedup vs baseline: 12.6919x; 12.6919x over previous
import functools

import numpy as np
import jax
import jax.numpy as jnp
from jax import lax
from jax.experimental import pallas as pl
from jax.experimental.pallas import tpu as pltpu
from jax.experimental.pallas import tpu_sc as plsc

B, N, C = 8, 4096, 64
M = 512
SAMPLES = (16, 32, 64)
R2 = tuple(np.float32(r * r) for r in (0.2, 0.4, 0.8))
DIMS = ((32, 32, 64), (64, 64, 128), (64, 96, 128))
D1S = (32, 64, 64)
MB_BQ = 256
F32 = jnp.float32


def _fps_body(pts_ref, cent_ref):
    x = pts_ref[:, 0, :]
    y = pts_ref[:, 1, :]
    z = pts_ref[:, 2, :]
    lane = lax.broadcasted_iota(jnp.int32, (B, N), 1)
    miota = lax.broadcasted_iota(jnp.int32, (B, M), 1)
    lx = x[:, 0:1]
    ly = y[:, 0:1]
    lz = z[:, 0:1]
    cx0 = jnp.broadcast_to(lx, (B, M))
    cy0 = jnp.broadcast_to(ly, (B, M))
    cz0 = jnp.broadcast_to(lz, (B, M))
    mind0 = jnp.full((B, N), jnp.inf, F32)

    def body(i, st):
        mind, lx, ly, lz, cx, cy, cz = st
        d = (x - lx) ** 2 + (y - ly) ** 2 + (z - lz) ** 2
        mind = jnp.minimum(mind, d)
        mx = jnp.max(mind, axis=1, keepdims=True)
        nxt = jnp.min(jnp.where(mind == mx, lane, N), axis=1, keepdims=True)
        sel = lane == nxt
        lx = jnp.sum(jnp.where(sel, x, 0.0), axis=1, keepdims=True)
        ly = jnp.sum(jnp.where(sel, y, 0.0), axis=1, keepdims=True)
        lz = jnp.sum(jnp.where(sel, z, 0.0), axis=1, keepdims=True)
        csel = miota == i
        cx = jnp.where(csel, lx, cx)
        cy = jnp.where(csel, ly, cy)
        cz = jnp.where(csel, lz, cz)
        return (mind, lx, ly, lz, cx, cy, cz)

    _, _, _, _, cx, cy, cz = lax.fori_loop(
        1, M, body, (mind0, lx, ly, lz, cx0, cy0, cz0))
    cent_ref[:, 0, :] = cx
    cent_ref[:, 1, :] = cy
    cent_ref[:, 2, :] = cz


def _fps(points):
    return pl.pallas_call(
        _fps_body,
        out_shape=jax.ShapeDtypeStruct((B, 3, M), F32),
    )(points)


def _bq_body(pts_ref, ct_ref, o16_ref, o32_ref, o64_ref):
    b = pl.program_id(0)
    px = pts_ref[0, 0, :].reshape(1, N)
    py = pts_ref[0, 1, :].reshape(1, N)
    pz = pts_ref[0, 2, :].reshape(1, N)
    cx = ct_ref[0, :, 0:1]
    cy = ct_ref[0, :, 1:2]
    cz = ct_ref[0, :, 2:3]
    d2 = (cx - px) ** 2 + (cy - py) ** 2 + (cz - pz) ** 2
    lane = lax.broadcasted_iota(jnp.int32, (MB_BQ, N), 1)
    for s, r2, oref in ((16, R2[0], o16_ref), (32, R2[1], o32_ref),
                        (64, R2[2], o64_ref)):
        key = jnp.where(d2 <= r2, lane, N)
        kiota = lax.broadcasted_iota(jnp.int32, (MB_BQ, s), 1)

        def step(k, st, key=key, kiota=kiota):
            prev, out = st
            nxt = jnp.min(jnp.where(key > prev, key, N), axis=1, keepdims=True)
            out = jnp.where(kiota == k, nxt, out)
            return (nxt, out)

        _, out = lax.fori_loop(
            0, s, step,
            (jnp.full((MB_BQ, 1), -1, jnp.int32),
             jnp.zeros((MB_BQ, s), jnp.int32)))
        first = out[:, 0:1]
        out = jnp.where(out == N, first, out)
        oref[0] = out + b * N


def _bq(points, cent_t):
    nmb = M // MB_BQ
    return pl.pallas_call(
        _bq_body,
        grid=(B, nmb),
        in_specs=[
            pl.BlockSpec((1, 3, N), lambda b, mb: (b, 0, 0)),
            pl.BlockSpec((1, MB_BQ, 3), lambda b, mb: (b, mb, 0)),
        ],
        out_specs=[
            pl.BlockSpec((1, MB_BQ, 16), lambda b, mb: (b, mb, 0)),
            pl.BlockSpec((1, MB_BQ, 32), lambda b, mb: (b, mb, 0)),
            pl.BlockSpec((1, MB_BQ, 64), lambda b, mb: (b, mb, 0)),
        ],
        out_shape=[
            jax.ShapeDtypeStruct((B, M, 16), jnp.int32),
            jax.ShapeDtypeStruct((B, M, 32), jnp.int32),
            jax.ShapeDtypeStruct((B, M, 64), jnp.int32),
        ],
    )(points, cent_t)


def _tables_body(xt_ref, wa_ref, ba_ref, wb_ref, bb_ref, wc_ref, bc_ref,
                 ta_ref, tb_ref, tc_ref):
    hi = lax.Precision.HIGHEST
    x = xt_ref[0]
    ta_ref[0] = jnp.dot(x, wa_ref[...], precision=hi) + ba_ref[...]
    tb_ref[0] = jnp.dot(x, wb_ref[...], precision=hi) + bb_ref[...]
    tc_ref[0] = jnp.dot(x, wc_ref[...], precision=hi) + bc_ref[...]


def _tables(xt, params):
    ws, bs = [], []
    for lyr in params:
        w1, b1 = lyr[0]
        ws.append(w1.T)
        bs.append(b1.reshape(1, -1))
    din = xt.shape[2]
    in_specs = [pl.BlockSpec((1, N, din), lambda b: (b, 0, 0))]
    args = [xt]
    for w, b_ in zip(ws, bs):
        in_specs.append(pl.BlockSpec(w.shape, lambda b: (0, 0)))
        in_specs.append(pl.BlockSpec(b_.shape, lambda b: (0, 0)))
        args.extend([w, b_])
    return pl.pallas_call(
        _tables_body,
        grid=(B,),
        in_specs=in_specs,
        out_specs=[pl.BlockSpec((1, N, d1), lambda b: (b, 0, 0))
                   for d1 in D1S],
        out_shape=[jax.ShapeDtypeStruct((B, N, d1), F32) for d1 in D1S],
    )(*args)


_GATHER_CHUNK = 512


def _make_gather():
    info = plsc.get_sparse_core_info()
    nc, ns = info.num_cores, info.num_subcores
    nw = nc * ns
    counts = tuple(B * M * s for s in SAMPLES)

    mesh = plsc.VectorSubcoreMesh(core_axis_name="c", subcore_axis_name="s")

    @functools.partial(
        pl.kernel,
        mesh=mesh,
        out_type=[jax.ShapeDtypeStruct((cnt, d1), F32)
                  for cnt, d1 in zip(counts, D1S)],
        scratch_types=[
            pltpu.VMEM((_GATHER_CHUNK,), jnp.int32),
            pltpu.VMEM((_GATHER_CHUNK, D1S[0]), F32),
            pltpu.VMEM((_GATHER_CHUNK, D1S[1]), F32),
            pltpu.SemaphoreType.DMA,
        ],
        compiler_params=pltpu.CompilerParams(use_tc_tiling_on_sc=False),
    )
    def gather_k(t1, t2, t3, i16, i32_, i64_, o16, o32_, o64_,
                 idx_v, rows1, rows2, sem):
        wid = lax.axis_index("s") * nc + lax.axis_index("c")
        for tbl, idx_hbm, out_hbm, rows_v, cnt in (
                (t1, i16, o16, rows1, counts[0]),
                (t2, i32_, o32_, rows2, counts[1]),
                (t3, i64_, o64_, rows2, counts[2])):
            rpw = cnt // nw
            base = wid * rpw

            def chunk(j, _, tbl=tbl, idx_hbm=idx_hbm, out_hbm=out_hbm,
                      rows_v=rows_v, base=base):
                off = base + j * _GATHER_CHUNK
                pltpu.sync_copy(idx_hbm.at[pl.ds(off, _GATHER_CHUNK)], idx_v)
                pltpu.async_copy(tbl.at[idx_v], rows_v, sem).wait()
                pltpu.sync_copy(rows_v, out_hbm.at[pl.ds(off, _GATHER_CHUNK)])
                return 0

            lax.fori_loop(0, rpw // _GATHER_CHUNK, chunk, 0)

    return gather_k


def _mlp_body(s, d1, d3, mb, g_ref, ct_ref, w1xt_ref,
              w2t_ref, b2_ref, w3t_ref, b3_ref, o_ref):
    hi = lax.Precision.HIGHEST
    q = jnp.dot(ct_ref[0], w1xt_ref[...], precision=hi)
    x1 = jnp.maximum(g_ref[0] - q.reshape(mb, 1, d1), 0.0)
    x1 = x1.reshape(mb * s, d1)
    x2 = jnp.maximum(jnp.dot(x1, w2t_ref[...], precision=hi) + b2_ref[...], 0.0)
    x3 = jnp.maximum(jnp.dot(x2, w3t_ref[...], precision=hi) + b3_ref[...], 0.0)
    o_ref[0] = jnp.max(x3.reshape(mb, s, d3), axis=1)


def _mlp(g, cent_t, layers, s, mb):
    (w1, b1), (w2, b2), (w3, b3) = layers
    d1, d2_, d3 = w1.shape[0], w2.shape[0], w3.shape[0]
    w1xt = w1[:, :3].T
    body = functools.partial(_mlp_body, s, d1, d3, mb)
    nmb = M // mb
    return pl.pallas_call(
        body,
        grid=(B, nmb),
        in_specs=[
            pl.BlockSpec((1, mb, s, d1), lambda b, i: (b, i, 0, 0)),
            pl.BlockSpec((1, mb, 3), lambda b, i: (b, i, 0)),
            pl.BlockSpec((3, d1), lambda b, i: (0, 0)),
            pl.BlockSpec((d1, d2_), lambda b, i: (0, 0)),
            pl.BlockSpec((1, d2_), lambda b, i: (0, 0)),
            pl.BlockSpec((d2_, d3), lambda b, i: (0, 0)),
            pl.BlockSpec((1, d3), lambda b, i: (0, 0)),
        ],
        out_specs=pl.BlockSpec((1, mb, d3), lambda b, i: (b, i, 0)),
        out_shape=jax.ShapeDtypeStruct((B, M, d3), F32),
    )(g.reshape(B, M, s, d1), cent_t, w1xt,
      w2.T, b2.reshape(1, d2_), w3.T, b3.reshape(1, d3))


def kernel(points, features, params):
    cent = _fps(points)
    cent_t = jnp.transpose(cent, (0, 2, 1))
    idx16, idx32, idx64 = _bq(points, cent_t)
    xt = jnp.concatenate(
        [jnp.transpose(points, (0, 2, 1)),
         jnp.transpose(features, (0, 2, 1))], axis=2)
    t1, t2, t3 = _tables(xt, params)
    gather_k = _make_gather()
    g16, g32, g64 = gather_k(
        t1.reshape(B * N, D1S[0]), t2.reshape(B * N, D1S[1]),
        t3.reshape(B * N, D1S[2]), idx16.reshape(-1), idx32.reshape(-1),
        idx64.reshape(-1))
    outs = []
    for g, layers, s, mb in ((g16, params[0], 16, 256),
                             (g32, params[1], 32, 256),
                             (g64, params[2], 64, 128)):
        o = _mlp(g, cent_t, layers, s, mb)
        outs.append(jnp.transpose(o, (0, 2, 1)))
    return (cent, jnp.concatenate(outs, axis=1))


# speedup vs baseline: 16.0179x; 1.2621x over previous
import functools

import numpy as np
import jax
import jax.numpy as jnp
from jax import lax
from jax.experimental import pallas as pl
from jax.experimental.pallas import tpu as pltpu
from jax.experimental.pallas import tpu_sc as plsc

B, N, C = 8, 4096, 64
M = 512
SAMPLES = (16, 32, 64)
R2 = tuple(np.float32(r * r) for r in (0.2, 0.4, 0.8))
DIMS = ((32, 32, 64), (64, 64, 128), (64, 96, 128))
D1S = (32, 64, 64)
MB_BQ = 256
F32 = jnp.float32


def _fps_body(pts_ref, cent_ref):
    x = pts_ref[:, 0, :]
    y = pts_ref[:, 1, :]
    z = pts_ref[:, 2, :]
    lane = lax.broadcasted_iota(jnp.int32, (B, N), 1)
    miota = lax.broadcasted_iota(jnp.int32, (B, M), 1)
    lx = x[:, 0:1]
    ly = y[:, 0:1]
    lz = z[:, 0:1]
    cx0 = jnp.broadcast_to(lx, (B, M))
    cy0 = jnp.broadcast_to(ly, (B, M))
    cz0 = jnp.broadcast_to(lz, (B, M))
    mind0 = jnp.full((B, N), jnp.inf, F32)

    def body(i, st):
        mind, lx, ly, lz, cx, cy, cz = st
        d = (x - lx) ** 2 + (y - ly) ** 2 + (z - lz) ** 2
        mind = jnp.minimum(mind, d)
        mx = jnp.max(mind, axis=1, keepdims=True)
        nxt = jnp.min(jnp.where(mind == mx, lane, N), axis=1, keepdims=True)
        sel = lane == nxt
        lx = jnp.sum(jnp.where(sel, x, 0.0), axis=1, keepdims=True)
        ly = jnp.sum(jnp.where(sel, y, 0.0), axis=1, keepdims=True)
        lz = jnp.sum(jnp.where(sel, z, 0.0), axis=1, keepdims=True)
        csel = miota == i
        cx = jnp.where(csel, lx, cx)
        cy = jnp.where(csel, ly, cy)
        cz = jnp.where(csel, lz, cz)
        return (mind, lx, ly, lz, cx, cy, cz)

    _, _, _, _, cx, cy, cz = lax.fori_loop(
        1, M, body, (mind0, lx, ly, lz, cx0, cy0, cz0))
    cent_ref[:, 0, :] = cx
    cent_ref[:, 1, :] = cy
    cent_ref[:, 2, :] = cz


def _fps(points):
    return pl.pallas_call(
        _fps_body,
        out_shape=jax.ShapeDtypeStruct((B, 3, M), F32),
    )(points)


def _bq_body(pts_ref, ct_ref, o16_ref, o32_ref, o64_ref):
    b = pl.program_id(0)
    px = pts_ref[0, 0, :].reshape(1, N)
    py = pts_ref[0, 1, :].reshape(1, N)
    pz = pts_ref[0, 2, :].reshape(1, N)
    cx = ct_ref[0, :, 0:1]
    cy = ct_ref[0, :, 1:2]
    cz = ct_ref[0, :, 2:3]
    d2 = (cx - px) ** 2 + (cy - py) ** 2 + (cz - pz) ** 2
    lane = lax.broadcasted_iota(jnp.int32, (MB_BQ, N), 1)
    for s, r2, oref in ((16, R2[0], o16_ref), (32, R2[1], o32_ref),
                        (64, R2[2], o64_ref)):
        key = jnp.where(d2 <= r2, lane, N)
        kiota = lax.broadcasted_iota(jnp.int32, (MB_BQ, s), 1)

        def step(k, st, key=key, kiota=kiota):
            prev, out = st
            nxt = jnp.min(jnp.where(key > prev, key, N), axis=1, keepdims=True)
            out = jnp.where(kiota == k, nxt, out)
            return (nxt, out)

        _, out = lax.fori_loop(
            0, s, step,
            (jnp.full((MB_BQ, 1), -1, jnp.int32),
             jnp.zeros((MB_BQ, s), jnp.int32)))
        first = out[:, 0:1]
        out = jnp.where(out == N, first, out)
        oref[0] = out + b * N


def _bq(points, cent_t):
    nmb = M // MB_BQ
    return pl.pallas_call(
        _bq_body,
        grid=(B, nmb),
        in_specs=[
            pl.BlockSpec((1, 3, N), lambda b, mb: (b, 0, 0)),
            pl.BlockSpec((1, MB_BQ, 3), lambda b, mb: (b, mb, 0)),
        ],
        out_specs=[
            pl.BlockSpec((1, MB_BQ, 16), lambda b, mb: (b, mb, 0)),
            pl.BlockSpec((1, MB_BQ, 32), lambda b, mb: (b, mb, 0)),
            pl.BlockSpec((1, MB_BQ, 64), lambda b, mb: (b, mb, 0)),
        ],
        out_shape=[
            jax.ShapeDtypeStruct((B, M, 16), jnp.int32),
            jax.ShapeDtypeStruct((B, M, 32), jnp.int32),
            jax.ShapeDtypeStruct((B, M, 64), jnp.int32),
        ],
    )(points, cent_t)


def _tables_body(xt_ref, wa_ref, ba_ref, wb_ref, bb_ref, wc_ref, bc_ref,
                 ta_ref, tb_ref, tc_ref):
    hi = lax.Precision.DEFAULT
    x = xt_ref[0]
    ta_ref[0] = jnp.dot(x, wa_ref[...], precision=hi) + ba_ref[...]
    tb_ref[0] = jnp.dot(x, wb_ref[...], precision=hi) + bb_ref[...]
    tc_ref[0] = jnp.dot(x, wc_ref[...], precision=hi) + bc_ref[...]


def _tables(xt, params):
    ws, bs = [], []
    for lyr in params:
        w1, b1 = lyr[0]
        ws.append(w1.T)
        bs.append(b1.reshape(1, -1))
    din = xt.shape[2]
    in_specs = [pl.BlockSpec((1, N, din), lambda b: (b, 0, 0))]
    args = [xt]
    for w, b_ in zip(ws, bs):
        in_specs.append(pl.BlockSpec(w.shape, lambda b: (0, 0)))
        in_specs.append(pl.BlockSpec(b_.shape, lambda b: (0, 0)))
        args.extend([w, b_])
    return pl.pallas_call(
        _tables_body,
        grid=(B,),
        in_specs=in_specs,
        out_specs=[pl.BlockSpec((1, N, d1), lambda b: (b, 0, 0))
                   for d1 in D1S],
        out_shape=[jax.ShapeDtypeStruct((B, N, d1), F32) for d1 in D1S],
    )(*args)


_GATHER_CHUNK = 512


def _make_gather():
    info = plsc.get_sparse_core_info()
    nc, ns = info.num_cores, info.num_subcores
    nw = nc * ns
    counts = tuple(B * M * s for s in SAMPLES)

    mesh = plsc.VectorSubcoreMesh(core_axis_name="c", subcore_axis_name="s")

    @functools.partial(
        pl.kernel,
        mesh=mesh,
        out_type=[jax.ShapeDtypeStruct((cnt, d1), F32)
                  for cnt, d1 in zip(counts, D1S)],
        scratch_types=[
            pltpu.VMEM((_GATHER_CHUNK,), jnp.int32),
            pltpu.VMEM((_GATHER_CHUNK, D1S[0]), F32),
            pltpu.VMEM((_GATHER_CHUNK, D1S[1]), F32),
            pltpu.SemaphoreType.DMA,
        ],
        compiler_params=pltpu.CompilerParams(use_tc_tiling_on_sc=False),
    )
    def gather_k(t1, t2, t3, i16, i32_, i64_, o16, o32_, o64_,
                 idx_v, rows1, rows2, sem):
        wid = lax.axis_index("s") * nc + lax.axis_index("c")
        for tbl, idx_hbm, out_hbm, rows_v, cnt in (
                (t1, i16, o16, rows1, counts[0]),
                (t2, i32_, o32_, rows2, counts[1]),
                (t3, i64_, o64_, rows2, counts[2])):
            rpw = cnt // nw
            base = wid * rpw

            def chunk(j, _, tbl=tbl, idx_hbm=idx_hbm, out_hbm=out_hbm,
                      rows_v=rows_v, base=base):
                off = base + j * _GATHER_CHUNK
                pltpu.sync_copy(idx_hbm.at[pl.ds(off, _GATHER_CHUNK)], idx_v)
                pltpu.async_copy(tbl.at[idx_v], rows_v, sem).wait()
                pltpu.sync_copy(rows_v, out_hbm.at[pl.ds(off, _GATHER_CHUNK)])
                return 0

            lax.fori_loop(0, rpw // _GATHER_CHUNK, chunk, 0)

    return gather_k


def _mlp_body(s, d1, d3, mb, g_ref, ct_ref, w1xt_ref,
              w2t_ref, b2_ref, w3t_ref, b3_ref, o_ref):
    hi = lax.Precision.DEFAULT
    q = jnp.dot(ct_ref[0], w1xt_ref[...], precision=hi)
    x1 = jnp.maximum(g_ref[0] - q.reshape(mb, 1, d1), 0.0)
    x1 = x1.reshape(mb * s, d1)
    x2 = jnp.maximum(jnp.dot(x1, w2t_ref[...], precision=hi) + b2_ref[...], 0.0)
    x3 = jnp.maximum(jnp.dot(x2, w3t_ref[...], precision=hi) + b3_ref[...], 0.0)
    o_ref[0] = jnp.max(x3.reshape(mb, s, d3), axis=1)


def _mlp(g, cent_t, layers, s, mb):
    (w1, b1), (w2, b2), (w3, b3) = layers
    d1, d2_, d3 = w1.shape[0], w2.shape[0], w3.shape[0]
    w1xt = w1[:, :3].T
    body = functools.partial(_mlp_body, s, d1, d3, mb)
    nmb = M // mb
    return pl.pallas_call(
        body,
        grid=(B, nmb),
        in_specs=[
            pl.BlockSpec((1, mb, s, d1), lambda b, i: (b, i, 0, 0)),
            pl.BlockSpec((1, mb, 3), lambda b, i: (b, i, 0)),
            pl.BlockSpec((3, d1), lambda b, i: (0, 0)),
            pl.BlockSpec((d1, d2_), lambda b, i: (0, 0)),
            pl.BlockSpec((1, d2_), lambda b, i: (0, 0)),
            pl.BlockSpec((d2_, d3), lambda b, i: (0, 0)),
            pl.BlockSpec((1, d3), lambda b, i: (0, 0)),
        ],
        out_specs=pl.BlockSpec((1, mb, d3), lambda b, i: (b, i, 0)),
        out_shape=jax.ShapeDtypeStruct((B, M, d3), F32),
    )(g.reshape(B, M, s, d1), cent_t, w1xt,
      w2.T, b2.reshape(1, d2_), w3.T, b3.reshape(1, d3))


def kernel(points, features, params):
    cent = _fps(points)
    cent_t = jnp.transpose(cent, (0, 2, 1))
    idx16, idx32, idx64 = _bq(points, cent_t)
    xt = jnp.concatenate(
        [jnp.transpose(points, (0, 2, 1)),
         jnp.transpose(features, (0, 2, 1))], axis=2)
    t1, t2, t3 = _tables(xt, params)
    gather_k = _make_gather()
    g16, g32, g64 = gather_k(
        t1.reshape(B * N, D1S[0]), t2.reshape(B * N, D1S[1]),
        t3.reshape(B * N, D1S[2]), idx16.reshape(-1), idx32.reshape(-1),
        idx64.reshape(-1))
    outs = []
    for g, layers, s, mb in ((g16, params[0], 16, 256),
                             (g32, params[1], 32, 256),
                             (g64, params[2], 64, 128)):
        o = _mlp(g, cent_t, layers, s, mb)
        outs.append(jnp.transpose(o, (0, 2, 1)))
    return (cent, jnp.concatenate(outs, axis=1))
